# Initial kernel scaffold; baseline (speedup 1.0000x reference)
import jax
import jax.numpy as jnp
from jax.experimental import pallas as pl


def kernel(x, edge_index, edge_attr, u, batch, W1, b1, W2, b2, Wg, bg):
    raise NotImplementedError("write your pallas kernel here")


# speedup vs baseline: 3.3121x; 3.3121x over previous
import functools

import jax
import jax.numpy as jnp
from jax import lax
from jax.experimental import pallas as pl
from jax.experimental.pallas import tpu as pltpu
from jax.experimental.pallas import tpu_sc as plsc

N = 10000
IN_CH = 128
EDGE_ATTRS = 16
OUT_CH = 128
GLOBAL_F = 32
B = 16

NC = 2
NS = 16
NW = NC * NS
CHUNK = 128

NP = 10112
NPT = NP // NS


def _sc_mesh():
    return plsc.VectorSubcoreMesh(core_axis_name="c", subcore_axis_name="s")


def _zero_vmem_rows(ref, nrows, width):
    z16 = jnp.zeros((16,), jnp.float32)

    def zrow(i, _):
        def zcol(j, _):
            ref[i, pl.ds(j * 16, 16)] = z16
            return 0
        return lax.fori_loop(0, width // 16, zcol, 0)
    lax.fori_loop(0, nrows, zrow, 0)


def _sc_gx_body(chunks_per_tile, col_hbm, row_hbm, x_hbm, gx_out,
                colv, rowv, rows_v, acc_gx, sem):
    cid = lax.axis_index("c")
    sid = lax.axis_index("s")
    wid = cid * NS + sid

    _zero_vmem_rows(rows_v, CHUNK, IN_CH)

    r0 = sid * NPT
    done = 0
    while done < NPT:
        nrows = min(CHUNK, NPT - done)
        pltpu.sync_copy(rows_v.at[pl.ds(0, nrows)],
                        acc_gx.at[pl.ds(r0 + done, nrows)])
        done += nrows
    plsc.subcore_barrier()

    ebase = wid * (chunks_per_tile * CHUNK)

    def chunk(k, _):
        off = ebase + k * CHUNK
        pltpu.sync_copy(col_hbm.at[pl.ds(off, CHUNK)], colv)
        pltpu.sync_copy(row_hbm.at[pl.ds(off, CHUNK)], rowv)
        pltpu.async_copy(x_hbm.at[colv], rows_v, sem).wait()
        pltpu.sync_copy(rows_v, acc_gx.at[rowv], add=True)
        return 0
    lax.fori_loop(0, chunks_per_tile, chunk, 0)
    plsc.subcore_barrier()

    done = 0
    while done < NPT:
        nrows = min(CHUNK, NPT - done)
        pltpu.sync_copy(acc_gx.at[pl.ds(r0 + done, nrows)],
                        gx_out.at[cid, pl.ds(r0 + done, nrows)])
        done += nrows


def _sc_ge_body(chunks_per_tile, row_hbm, attr_hbm, ge_out,
                rowv, attr_v, big_v, acc_ge, sem):
    cid = lax.axis_index("c")
    sid = lax.axis_index("s")
    wid = cid * NS + sid

    _zero_vmem_rows(big_v, CHUNK, IN_CH)

    r0 = sid * NPT
    done = 0
    while done < NPT:
        nrows = min(CHUNK, NPT - done)
        pltpu.sync_copy(big_v.at[pl.ds(0, nrows)],
                        acc_ge.at[pl.ds(r0 + done, nrows)])
        done += nrows

    o16 = jnp.ones((16,), jnp.float32)

    def orow(i, _):
        big_v[i, pl.ds(EDGE_ATTRS, 16)] = o16
        return 0
    lax.fori_loop(0, CHUNK, orow, 0)
    plsc.subcore_barrier()

    ebase = wid * (chunks_per_tile * CHUNK)

    def chunk(k, _):
        off = ebase + k * CHUNK
        pltpu.sync_copy(row_hbm.at[pl.ds(off, CHUNK)], rowv)
        pltpu.sync_copy(attr_hbm.at[pl.ds(off, CHUNK)], attr_v)

        def crow(i, _):
            big_v[i, pl.ds(0, EDGE_ATTRS)] = attr_v[i, pl.ds(0, EDGE_ATTRS)]
            return 0
        lax.fori_loop(0, CHUNK, crow, 0)
        pltpu.sync_copy(big_v, acc_ge.at[rowv], add=True)
        return 0
    lax.fori_loop(0, chunks_per_tile, chunk, 0)
    plsc.subcore_barrier()

    done = 0
    while done < NPT:
        nrows = min(CHUNK, NPT - done)
        pltpu.sync_copy(acc_ge.at[pl.ds(r0 + done, nrows)],
                        ge_out.at[cid, pl.ds(r0 + done, nrows)])
        done += nrows


def _make_sc_gx_call(chunks_per_tile):
    return pl.kernel(
        functools.partial(_sc_gx_body, chunks_per_tile),
        out_type=jax.ShapeDtypeStruct((NC, NP, IN_CH), jnp.float32),
        mesh=_sc_mesh(),
        scratch_types=[
            pltpu.VMEM((CHUNK,), jnp.int32),
            pltpu.VMEM((CHUNK,), jnp.int32),
            pltpu.VMEM((CHUNK, IN_CH), jnp.float32),
            pltpu.VMEM_SHARED((NP, IN_CH), jnp.float32),
            pltpu.SemaphoreType.DMA,
        ],
    )


def _make_sc_ge_call(chunks_per_tile):
    return pl.kernel(
        functools.partial(_sc_ge_body, chunks_per_tile),
        out_type=jax.ShapeDtypeStruct((NC, NP, IN_CH), jnp.float32),
        mesh=_sc_mesh(),
        scratch_types=[
            pltpu.VMEM((CHUNK,), jnp.int32),
            pltpu.VMEM((CHUNK, EDGE_ATTRS), jnp.float32),
            pltpu.VMEM((CHUNK, IN_CH), jnp.float32),
            pltpu.VMEM_SHARED((NP, IN_CH), jnp.float32),
            pltpu.SemaphoreType.DMA,
        ],
    )


ROWS_BLK = 1000
GRID = N // ROWS_BLK


def _tc_body(gxp, gep, batchr, w1a, w1b, b1r, w2a, w2b, b2r,
             ur, wga, wgb, bgr, xn_out, g_out, acc_sx, acc_cb):
    i = pl.program_id(0)

    @pl.when(i == 0)
    def _():
        acc_sx[...] = jnp.zeros_like(acc_sx)
        acc_cb[...] = jnp.zeros_like(acc_cb)

    gx = gxp[0] + gxp[1]
    gec = gep[0] + gep[1]
    ge = gec[:, :EDGE_ATTRS]
    cnt = gec[:, EDGE_ATTRS:EDGE_ATTRS + 1]
    cntc = jnp.maximum(cnt, 1.0)

    sumh = (jnp.dot(gx, w1a[...], preferred_element_type=jnp.float32)
            + jnp.dot(ge, w1b[...], preferred_element_type=jnp.float32)
            + cnt * b1r[...])
    agg = sumh / cntc

    bvec = batchr[...][:, 0]
    onehot = (bvec[:, None] == lax.broadcasted_iota(jnp.int32, (1, B), 1)
              ).astype(jnp.float32)
    ub = jnp.dot(onehot, ur[...], preferred_element_type=jnp.float32)
    xn = (jnp.dot(agg, w2a[...], preferred_element_type=jnp.float32)
          + jnp.dot(ub, w2b[...], preferred_element_type=jnp.float32)
          + b2r[...])
    xn_out[...] = xn

    acc_sx[...] += lax.dot_general(onehot, xn, (((0,), (0,)), ((), ())),
                                   preferred_element_type=jnp.float32)
    acc_cb[...] += jnp.broadcast_to(
        jnp.sum(onehot, axis=0)[:, None], (B, OUT_CH))

    @pl.when(i == GRID - 1)
    def _():
        mean_xn = acc_sx[...] / jnp.maximum(acc_cb[...], 1.0)
        g_out[...] = (jnp.dot(ur[...], wga[...],
                              preferred_element_type=jnp.float32)
                      + jnp.dot(mean_xn, wgb[...],
                                preferred_element_type=jnp.float32)
                      + bgr[...])


def _tc_call(gx_p, ge_p, batch2d, w1a, w1b, b1r, w2a, w2b, b2r,
             u, wga, wgb, bgr):
    full = lambda shape: pl.BlockSpec(shape, lambda i: (0,) * len(shape))
    return pl.pallas_call(
        _tc_body,
        grid=(GRID,),
        in_specs=[
            pl.BlockSpec((NC, ROWS_BLK, IN_CH), lambda i: (0, i, 0)),
            pl.BlockSpec((NC, ROWS_BLK, IN_CH), lambda i: (0, i, 0)),
            pl.BlockSpec((ROWS_BLK, 1), lambda i: (i, 0)),
            full((IN_CH, OUT_CH)),
            full((EDGE_ATTRS, OUT_CH)),
            full((1, OUT_CH)),
            full((OUT_CH, OUT_CH)),
            full((GLOBAL_F, OUT_CH)),
            full((1, OUT_CH)),
            full((B, GLOBAL_F)),
            full((GLOBAL_F, GLOBAL_F)),
            full((OUT_CH, GLOBAL_F)),
            full((1, GLOBAL_F)),
        ],
        out_specs=[
            pl.BlockSpec((ROWS_BLK, OUT_CH), lambda i: (i, 0)),
            pl.BlockSpec((B, GLOBAL_F), lambda i: (0, 0)),
        ],
        out_shape=[
            jax.ShapeDtypeStruct((N, OUT_CH), jnp.float32),
            jax.ShapeDtypeStruct((B, GLOBAL_F), jnp.float32),
        ],
        scratch_shapes=[
            pltpu.VMEM((B, OUT_CH), jnp.float32),
            pltpu.VMEM((B, OUT_CH), jnp.float32),
        ],
    )(gx_p, ge_p, batch2d, w1a, w1b, b1r, w2a, w2b, b2r,
      u, wga, wgb, bgr)


@jax.jit
def kernel(x, edge_index, edge_attr, u, batch, W1, b1, W2, b2, Wg, bg):
    E = edge_attr.shape[0]
    row = edge_index[0].astype(jnp.int32)
    col = edge_index[1].astype(jnp.int32)

    per_tile = -(-E // (NW * CHUNK)) * CHUNK
    e_pad = per_tile * NW
    pad = e_pad - E
    colp = jnp.concatenate([col, jnp.zeros((pad,), jnp.int32)])
    rowp = jnp.concatenate(
        [row, N + (jnp.arange(pad, dtype=jnp.int32) % 16)])
    attrp = jnp.concatenate(
        [edge_attr, jnp.zeros((pad, EDGE_ATTRS), jnp.float32)])

    cpt = per_tile // CHUNK
    gx_p = _make_sc_gx_call(cpt)(colp, rowp, x)
    ge_p = _make_sc_ge_call(cpt)(rowp, attrp)

    batch2d = batch.astype(jnp.int32)[:, None]
    xn, g = _tc_call(
        gx_p, ge_p, batch2d,
        W1[:IN_CH], W1[IN_CH:], b1[None, :],
        W2[:OUT_CH], W2[OUT_CH:], b2[None, :],
        u, Wg[:GLOBAL_F], Wg[GLOBAL_F:], bg[None, :])
    return (xn, edge_attr, g)
